# Initial kernel scaffold; baseline (speedup 1.0000x reference)
import jax
import jax.numpy as jnp
from jax.experimental import pallas as pl


def kernel(boxes, scores):
    raise NotImplementedError("write your pallas kernel here")


# speedup vs baseline: 52.1739x; 52.1739x over previous
import jax
import jax.numpy as jnp
from jax import lax
from jax.experimental import pallas as pl

N = 5000
B = 128
NB = 40
NPAD = NB * B
THR = 0.5


def _nms_body(xr, yr, rr, br, xc, yc, rc, bc, keep_ref):
    b = pl.program_id(0)

    @pl.when(b == 0)
    def _init():
        keep_ref[...] = jnp.ones((NB, B), jnp.float32)

    xi = xc[...]
    yi = yc[...]
    ri = rc[...]
    bi = bc[...]
    ai = (ri - xi + 1.0) * (bi - yi + 1.0)

    def supp_mat(xj, yj, rj, bj):
        aj = (rj - xj + 1.0) * (bj - yj + 1.0)
        cw = jnp.minimum(ri, rj) - jnp.maximum(xi, xj) + 1.0
        ch = jnp.minimum(bi, bj) - jnp.maximum(yi, yj) + 1.0
        cross = jnp.maximum(cw, 0.0) * jnp.maximum(ch, 0.0)
        union = ai + aj - cross
        return (cross > THR * (union + 1e-6)).astype(jnp.float32)

    xj = xr[pl.ds(b, 1), :]
    yj = yr[pl.ds(b, 1), :]
    rj = rr[pl.ds(b, 1), :]
    bj = br[pl.ds(b, 1), :]
    s_bb = supp_mat(xj, yj, rj, bj)
    ii = lax.broadcasted_iota(jnp.int32, (B, B), 0)
    jj = lax.broadcasted_iota(jnp.int32, (B, B), 1)
    s_bb = s_bb * (ii < jj).astype(jnp.float32)

    init = keep_ref[pl.ds(b, 1), :]

    def cond(c):
        return c[1]

    def body(c):
        keep, _ = c
        cnt = lax.dot_general(keep, s_bb, (((1,), (0,)), ((), ())),
                              preferred_element_type=jnp.float32)
        knew = init * (cnt < 0.5).astype(jnp.float32)
        return knew, jnp.any(knew != keep)

    keep_b, _ = lax.while_loop(cond, body, (init, True))
    keep_ref[pl.ds(b, 1), :] = keep_b

    def tail(t, carry):
        xt = xr[pl.ds(t, 1), :]
        yt = yr[pl.ds(t, 1), :]
        rt = rr[pl.ds(t, 1), :]
        bt = br[pl.ds(t, 1), :]
        s_bt = supp_mat(xt, yt, rt, bt)
        cnt = lax.dot_general(keep_b, s_bt, (((1,), (0,)), ((), ())),
                              preferred_element_type=jnp.float32)
        keep_ref[pl.ds(t, 1), :] = keep_ref[pl.ds(t, 1), :] * (
            cnt < 0.5).astype(jnp.float32)
        return carry

    lax.fori_loop(b + 1, NB, tail, 0)


def _nms_sorted(xs, ys, rs, bs, interpret=False):
    full = pl.BlockSpec((NB, B), lambda b: (0, 0))
    col = pl.BlockSpec((B, 1), lambda b: (b, 0))
    keep = pl.pallas_call(
        _nms_body,
        grid=(NB,),
        in_specs=[full, full, full, full, col, col, col, col],
        out_specs=pl.BlockSpec((NB, B), lambda b: (0, 0)),
        out_shape=jax.ShapeDtypeStruct((NB, B), jnp.float32),
        interpret=interpret,
    )(xs.reshape(NB, B), ys.reshape(NB, B), rs.reshape(NB, B),
      bs.reshape(NB, B), xs.reshape(NPAD, 1), ys.reshape(NPAD, 1),
      rs.reshape(NPAD, 1), bs.reshape(NPAD, 1))
    return keep.reshape(NPAD)


def kernel(boxes, scores, interpret=False):
    order = jnp.argsort(-scores)
    boxes_s = boxes[order]
    pad = NPAD - N
    far = jnp.arange(pad, dtype=jnp.float32) * 1000.0 + 1.0e7
    xs = jnp.concatenate([boxes_s[:, 0], far])
    ys = jnp.concatenate([boxes_s[:, 1], far])
    rs = jnp.concatenate([boxes_s[:, 2], far + 1.0])
    bs = jnp.concatenate([boxes_s[:, 3], far + 1.0])
    keep_sorted = _nms_sorted(xs, ys, rs, bs, interpret=interpret)[:N]
    keep = jnp.zeros((N,), jnp.float32).at[order].set(keep_sorted)
    return scores * keep
